# baseline (speedup 1.0000x reference)
import jax
import jax.numpy as jnp
from jax.experimental import pallas as pl

_DEPTH = 1000
_LANES = 128
_PERIOD = 125
_S = 4000


def _patterns():
    f = jnp.arange(_PERIOD * _LANES, dtype=jnp.int32).reshape(_PERIOD, _LANES)
    d = f % _DEPTH
    p = f // _DEPTH
    p0 = (jnp.arange(_PERIOD, dtype=jnp.int32) * _LANES) // _DEPTH
    in_a = p == p0[:, None]
    dpa = jnp.where(in_a, d, -1)
    dpb = jnp.where(~in_a, d, -1)
    return dpa, dpb


def _onehot_block(dpa_ref, dpb_ref, ia_ref, ib_ref, out_ref):
    reps = _S // _PERIOD
    dpa = jnp.tile(dpa_ref[...], (reps, 1))
    dpb = jnp.tile(dpb_ref[...], (reps, 1))
    ia = ia_ref[...]
    ib = ib_ref[...]
    mask = (dpa == ia) | (dpb == ib)
    out_ref[...] = jnp.where(mask, jnp.float32(1.0), jnp.float32(0.0))


def _row_select_mats():
    t = (jnp.arange(_PERIOD) * _LANES) // _DEPTH
    g = _PERIOD * _LANES // _DEPTH
    ma = (t[None, :] == jnp.arange(g)[:, None]).astype(jnp.float32)
    tb = jnp.minimum(t + 1, g - 1)
    mb = (tb[None, :] == jnp.arange(g)[:, None]).astype(jnp.float32)
    return ma, mb


def kernel(indices):
    B, P = indices.shape
    n_rows = B * P
    n_flat = n_rows * _DEPTH // _LANES
    g = _PERIOD * _LANES // _DEPTH
    idx_g = indices.astype(jnp.float32).reshape(-1, g)
    ma, mb = _row_select_mats()
    hi = jax.lax.Precision.HIGHEST
    idx_a = jnp.rint(jnp.matmul(idx_g, ma, precision=hi)).astype(
        jnp.int32).reshape(n_flat, 1)
    idx_b = jnp.rint(jnp.matmul(idx_g, mb, precision=hi)).astype(
        jnp.int32).reshape(n_flat, 1)
    dpa, dpb = _patterns()

    out2 = pl.pallas_call(
        _onehot_block,
        grid=(n_flat // _S,),
        in_specs=[
            pl.BlockSpec((_PERIOD, _LANES), lambda i: (0, 0)),
            pl.BlockSpec((_PERIOD, _LANES), lambda i: (0, 0)),
            pl.BlockSpec((_S, 1), lambda i: (i, 0)),
            pl.BlockSpec((_S, 1), lambda i: (i, 0)),
        ],
        out_specs=pl.BlockSpec((_S, _LANES), lambda i: (i, 0)),
        out_shape=jax.ShapeDtypeStruct((n_flat, _LANES), jnp.float32),
    )(dpa, dpb, idx_a, idx_b)
    return out2.reshape(B, P, _DEPTH)


# speedup vs baseline: 3.3979x; 3.3979x over previous
import functools

import jax
import jax.numpy as jnp
from jax.experimental import pallas as pl
from jax.experimental.pallas import tpu as pltpu

_DEPTH = 1000
_BB = 32
_NBUF = 4


def _onehot_body(nblk, idx_ref, out_hbm, *scratch):
    bufs, sems = scratch[:_NBUF], scratch[_NBUF]
    i = pl.program_id(0)
    slot = jax.lax.rem(i, _NBUF)

    for s in range(_NBUF):
        @pl.when(jnp.logical_and(slot == s, i >= _NBUF))
        def _wait_prev(s=s):
            pltpu.make_async_copy(
                bufs[s],
                out_hbm.at[pl.ds((i - _NBUF) * _BB, _BB)],
                sems.at[s],
            ).wait()

    idx = idx_ref[...]
    iota = jax.lax.broadcasted_iota(
        jnp.int32, (_BB, idx.shape[1], _DEPTH), 2)
    val = jnp.where(idx[..., None] == iota, jnp.float32(1.0),
                    jnp.float32(0.0))

    for s in range(_NBUF):
        @pl.when(slot == s)
        def _emit(s=s):
            bufs[s][...] = val
            pltpu.make_async_copy(
                bufs[s],
                out_hbm.at[pl.ds(i * _BB, _BB)],
                sems.at[s],
            ).start()

    @pl.when(i == nblk - 1)
    def _drain():
        base = nblk - _NBUF
        for s in range(_NBUF):
            step = base + ((s - base) % _NBUF)
            pltpu.make_async_copy(
                bufs[step % _NBUF],
                out_hbm.at[pl.ds(step * _BB, _BB)],
                sems.at[step % _NBUF],
            ).wait()


def kernel(indices):
    B, P = indices.shape
    indices = indices.astype(jnp.int32)
    nblk = B // _BB
    return pl.pallas_call(
        functools.partial(_onehot_body, nblk),
        grid=(nblk,),
        in_specs=[pl.BlockSpec((_BB, P), lambda i: (i, 0))],
        out_specs=pl.BlockSpec(memory_space=pltpu.MemorySpace.HBM),
        out_shape=jax.ShapeDtypeStruct((B, P, _DEPTH), jnp.float32),
        scratch_shapes=(
            [pltpu.VMEM((_BB, P, _DEPTH), jnp.float32) for _ in range(_NBUF)]
            + [pltpu.SemaphoreType.DMA((_NBUF,))]
        ),
    )(indices)


# speedup vs baseline: 3.4012x; 1.0010x over previous
import jax
import jax.numpy as jnp
from jax.experimental import pallas as pl

_DEPTH = 1000
_BB = 64


def _onehot_block(idx_ref, out_ref):
    idx = idx_ref[...]
    iota = jax.lax.broadcasted_iota(jnp.int32, out_ref.shape, 2)
    out_ref[...] = jnp.where(idx[..., None] == iota, jnp.float32(1.0),
                             jnp.float32(0.0))


def kernel(indices):
    B, P = indices.shape
    indices = indices.astype(jnp.int32)
    return pl.pallas_call(
        _onehot_block,
        grid=(B // _BB,),
        in_specs=[pl.BlockSpec((_BB, P), lambda i: (i, 0))],
        out_specs=pl.BlockSpec((_BB, P, _DEPTH), lambda i: (i, 0, 0)),
        out_shape=jax.ShapeDtypeStruct((B, P, _DEPTH), jnp.float32),
    )(indices)
